# baseline (speedup 1.0000x reference)
import functools

import jax
import jax.numpy as jnp
from jax import lax
from jax.experimental import pallas as pl
from jax.experimental.pallas import tpu as pltpu
from jax.experimental.pallas import tpu_sc as plsc

G = 4
N = 10000
E = 320000
D = 128
OUT = 64
NC = 2
NS = 16
EPT = E // NS
C = 125
NCH = EPT // C
RP = N // NS
BN = 2000


def _sc_mesh():
    return plsc.VectorSubcoreMesh(core_axis_name="c", subcore_axis_name="s",
                                  num_cores=NC, num_subcores=NS)


def _make_hist():
    @functools.partial(
        pl.kernel,
        out_type=jax.ShapeDtypeStruct((G, N, 16), jnp.float32),
        mesh=_sc_mesh(),
        scratch_types=[
            pltpu.VMEM_SHARED((N, 16), jnp.float32),
            pltpu.VMEM((NCH, C), jnp.int32),
            pltpu.VMEM((C, 16), jnp.float32),
        ],
        compiler_params=pltpu.CompilerParams(use_tc_tiling_on_sc=False),
    )
    def hist(dstm, z_ref, o_ref, deg_ref, accd, dstv, onesv):
        c = lax.axis_index("c")
        s = lax.axis_index("s")
        pltpu.sync_copy(o_ref, onesv)
        for gs in range(2):
            g = c * 2 + gs
            pltpu.sync_copy(z_ref.at[pl.ds(s * RP, RP)],
                            accd.at[pl.ds(s * RP, RP)])
            pltpu.sync_copy(dstm.at[g, s], dstv)
            plsc.subcore_barrier()

            @pl.loop(0, NCH)
            def _chunk(j):
                pltpu.sync_copy(onesv, accd.at[dstv.at[j]], add=True)

            plsc.subcore_barrier()
            pltpu.sync_copy(accd.at[pl.ds(s * RP, RP)],
                            deg_ref.at[g, pl.ds(s * RP, RP)])

    return hist


def _make_prop():
    @functools.partial(
        pl.kernel,
        out_type=jax.ShapeDtypeStruct((G, N, D), jnp.float32),
        mesh=_sc_mesh(),
        scratch_types=[
            pltpu.VMEM_SHARED((N, D), jnp.float32),
            pltpu.VMEM((12, 2, C), jnp.int32),
            pltpu.VMEM((3, C, D), jnp.float32),
            pltpu.SemaphoreType.DMA((12,)),
            pltpu.SemaphoreType.DMA((3,)),
            pltpu.SemaphoreType.DMA((3,)),
        ],
        compiler_params=pltpu.CompilerParams(use_tc_tiling_on_sc=False),
    )
    def prop(p_ref, sdm, out_ref, acc, ibuf, rows, semi, semg, sems):
        c = lax.axis_index("c")
        s = lax.axis_index("s")
        for gs in range(2):
            g = c * 2 + gs
            pltpu.sync_copy(p_ref.at[pl.ds(g * N + s * RP, RP)],
                            acc.at[pl.ds(s * RP, RP)])
            for b in range(6):
                pltpu.async_copy(sdm.at[g, s, b], ibuf.at[b], semi.at[b])
            plsc.subcore_barrier()

            @pl.loop(0, NCH)
            def _chunk(j):
                b3 = lax.rem(j, 3)
                b12 = lax.rem(j, 12)
                b12n = lax.rem(j + 6, 12)

                @pl.when(j >= 3)
                def _():
                    b12d = lax.rem(j + 9, 12)
                    pltpu.make_async_copy(rows.at[b3],
                                          acc.at[ibuf.at[b12d, 1]],
                                          sems.at[b3]).wait()

                pltpu.make_async_copy(sdm.at[g, s, j], ibuf.at[b12],
                                      semi.at[b12]).wait()
                pltpu.async_copy(p_ref.at[ibuf.at[b12, 0]], rows.at[b3],
                                 semg.at[b3])

                @pl.when(j + 6 < NCH)
                def _():
                    pltpu.async_copy(sdm.at[g, s, j + 6], ibuf.at[b12n],
                                     semi.at[b12n])

                @pl.when(j >= 2)
                def _():
                    b3p = lax.rem(j + 1, 3)
                    b12p = lax.rem(j + 10, 12)
                    pltpu.make_async_copy(p_ref.at[ibuf.at[b12p, 0]],
                                          rows.at[b3p], semg.at[b3p]).wait()
                    pltpu.async_copy(rows.at[b3p], acc.at[ibuf.at[b12p, 1]],
                                     sems.at[b3p], add=True)

            for k in (NCH - 2, NCH - 1):
                pltpu.make_async_copy(p_ref.at[ibuf.at[k % 12, 0]],
                                      rows.at[k % 3], semg.at[k % 3]).wait()
                pltpu.async_copy(rows.at[k % 3], acc.at[ibuf.at[k % 12, 1]],
                                 sems.at[k % 3], add=True)
            for k in (NCH - 3, NCH - 2, NCH - 1):
                pltpu.make_async_copy(rows.at[k % 3],
                                      acc.at[ibuf.at[k % 12, 1]],
                                      sems.at[k % 3]).wait()
            plsc.subcore_barrier()
            pltpu.sync_copy(acc.at[pl.ds(s * RP, RP)],
                            out_ref.at[g, pl.ds(s * RP, RP)])

    return prop


def _tc1(x, w1, deg16):
    def body(x_ref, w_ref, deg_ref, p_ref):
        dinv = lax.rsqrt(deg_ref[0][:, 0:1] + 1.0)
        p_ref[...] = jnp.dot(x_ref[0].astype(jnp.bfloat16),
                             w_ref[0].astype(jnp.bfloat16),
                             preferred_element_type=jnp.float32) * dinv

    return pl.pallas_call(
        body,
        grid=(G,),
        in_specs=[
            pl.BlockSpec((1, N, D), lambda g: (g, 0, 0)),
            pl.BlockSpec((1, D, D), lambda g: (g, 0, 0)),
            pl.BlockSpec((1, N, 16), lambda g: (g, 0, 0)),
        ],
        out_specs=pl.BlockSpec((N, D), lambda g: (g, 0)),
        out_shape=jax.ShapeDtypeStruct((G * N, D), jnp.float32),
    )(x, w1, deg16)


def _tc2(s1, deg16, b1, wml):
    def body(s_ref, deg_ref, b_ref, w_ref, p_ref):
        dinv = lax.rsqrt(deg_ref[0][:, 0:1] + 1.0)
        h = jnp.maximum(s_ref[0] * dinv + b_ref[0], 0.0)
        p_ref[...] = jnp.dot(h.astype(jnp.bfloat16),
                             w_ref[0].astype(jnp.bfloat16),
                             preferred_element_type=jnp.float32) * dinv

    return pl.pallas_call(
        body,
        grid=(G,),
        in_specs=[
            pl.BlockSpec((1, N, D), lambda g: (g, 0, 0)),
            pl.BlockSpec((1, N, 16), lambda g: (g, 0, 0)),
            pl.BlockSpec((1, 1, D), lambda g: (g, 0, 0)),
            pl.BlockSpec((1, D, D), lambda g: (g, 0, 0)),
        ],
        out_specs=pl.BlockSpec((N, D), lambda g: (g, 0)),
        out_shape=jax.ShapeDtypeStruct((G * N, D), jnp.float32),
    )(s1, deg16, b1, wml)


def _tc3(s2, deg16, bml):
    def body(s_ref, deg_ref, b_ref, out_ref):
        dinv = lax.rsqrt(deg_ref[0][:, 0:1] + 1.0)
        t = s_ref[0] * dinv + b_ref[0]
        out_ref[0, 0] = t[:, :OUT]
        out_ref[1, 0] = t[:, OUT:]

    return pl.pallas_call(
        body,
        grid=(G, N // BN),
        in_specs=[
            pl.BlockSpec((1, BN, D), lambda g, i: (g, i, 0)),
            pl.BlockSpec((1, BN, 16), lambda g, i: (g, i, 0)),
            pl.BlockSpec((1, 1, D), lambda g, i: (g, 0, 0)),
        ],
        out_specs=pl.BlockSpec((2, 1, BN, OUT), lambda g, i: (0, g, i, 0)),
        out_shape=jax.ShapeDtypeStruct((2, G, N, OUT), jnp.float32),
    )(s2, deg16, bml)


def kernel(x, edge_index, W1, b1, Wmu, bmu, Wls, bls):
    ei = edge_index.astype(jnp.int32)
    offs = (jnp.arange(G, dtype=jnp.int32) * N)[:, None]
    src = (ei[:, 0, :] + offs).reshape(G, NS, EPT)
    dst = ei[:, 1, :].reshape(G, NS, EPT)
    srcm = src.reshape(G, NS, NCH, C)
    dstm = dst.reshape(G, NS, NCH, C)
    sdm = jnp.stack([srcm, dstm], axis=3)

    zeros16 = jnp.zeros((N, 16), jnp.float32)
    ones16 = jnp.ones((C, 16), jnp.float32)

    hist = _make_hist()
    prop = _make_prop()

    deg16 = hist(dstm, zeros16, ones16)
    p1 = _tc1(x, W1, deg16)
    s1 = prop(p1, sdm)
    wml = jnp.concatenate([Wmu, Wls], axis=2)
    bml = jnp.concatenate([bmu, bls], axis=1)[:, None, :]
    p2 = _tc2(s1, deg16, b1[:, None, :], wml)
    s2 = prop(p2, sdm)
    return _tc3(s2, deg16, bml)


# speedup vs baseline: 1.0126x; 1.0126x over previous
import functools

import jax
import jax.numpy as jnp
from jax import lax
from jax.experimental import pallas as pl
from jax.experimental.pallas import tpu as pltpu
from jax.experimental.pallas import tpu_sc as plsc

G = 4
N = 10000
E = 320000
D = 128
OUT = 64
NC = 2
NS = 16
EPT = E // NS
C = 125
NCH = EPT // C
CP = 80
NCP = EPT // CP
RP = N // NS
BN = 2000


def _sc_mesh():
    return plsc.VectorSubcoreMesh(core_axis_name="c", subcore_axis_name="s",
                                  num_cores=NC, num_subcores=NS)


def _make_hist():
    @functools.partial(
        pl.kernel,
        out_type=jax.ShapeDtypeStruct((G, N, 16), jnp.float32),
        mesh=_sc_mesh(),
        scratch_types=[
            pltpu.VMEM_SHARED((N, 16), jnp.float32),
            pltpu.VMEM((NCH, C), jnp.int32),
            pltpu.VMEM((C, 16), jnp.float32),
        ],
        compiler_params=pltpu.CompilerParams(use_tc_tiling_on_sc=False),
    )
    def hist(dstm, z_ref, o_ref, deg_ref, accd, dstv, onesv):
        c = lax.axis_index("c")
        s = lax.axis_index("s")
        pltpu.sync_copy(o_ref, onesv)
        for gs in range(2):
            g = c * 2 + gs
            pltpu.sync_copy(z_ref.at[pl.ds(s * RP, RP)],
                            accd.at[pl.ds(s * RP, RP)])
            pltpu.sync_copy(dstm.at[g, s], dstv)
            plsc.subcore_barrier()

            @pl.loop(0, NCH)
            def _chunk(j):
                pltpu.sync_copy(onesv, accd.at[dstv.at[j]], add=True)

            plsc.subcore_barrier()
            pltpu.sync_copy(accd.at[pl.ds(s * RP, RP)],
                            deg_ref.at[g, pl.ds(s * RP, RP)])

    return hist


def _make_prop():
    @functools.partial(
        pl.kernel,
        out_type=jax.ShapeDtypeStruct((G, N, D), jnp.float32),
        mesh=_sc_mesh(),
        scratch_types=[
            pltpu.VMEM_SHARED((N, D), jnp.float32),
            pltpu.VMEM((16, CP), jnp.int32),
            pltpu.VMEM((16, CP), jnp.int32),
            pltpu.VMEM((4, CP, D), jnp.float32),
            pltpu.SemaphoreType.DMA((16,)),
            pltpu.SemaphoreType.DMA((4,)),
            pltpu.SemaphoreType.DMA((4,)),
        ],
        compiler_params=pltpu.CompilerParams(use_tc_tiling_on_sc=False),
    )
    def prop(p_ref, srco, ei, out_ref, acc, ibs, ibd, rows, semi, semg, sems):
        c = lax.axis_index("c")
        s = lax.axis_index("s")

        def fire_idx(g, k, slot):
            base = s * EPT + k * CP
            pltpu.async_copy(srco.at[g, pl.ds(base, CP)], ibs.at[slot],
                             semi.at[slot])
            pltpu.async_copy(ei.at[g, 1, pl.ds(base, CP)], ibd.at[slot],
                             semi.at[slot])

        def wait_idx(g, k, slot):
            base = s * EPT + k * CP
            pltpu.make_async_copy(srco.at[g, pl.ds(base, CP)], ibs.at[slot],
                                  semi.at[slot]).wait()
            pltpu.make_async_copy(ei.at[g, 1, pl.ds(base, CP)], ibd.at[slot],
                                  semi.at[slot]).wait()

        for gs in range(2):
            g = c * 2 + gs
            pltpu.sync_copy(p_ref.at[pl.ds(g * N + s * RP, RP)],
                            acc.at[pl.ds(s * RP, RP)])
            for b in range(6):
                fire_idx(g, b, b)
            plsc.subcore_barrier()

            @pl.loop(0, NCP)
            def _chunk(j):
                b4 = lax.rem(j, 4)
                b16 = lax.rem(j, 16)

                @pl.when(j >= 4)
                def _():
                    b16d = lax.rem(j + 12, 16)
                    pltpu.make_async_copy(rows.at[b4],
                                          acc.at[ibd.at[b16d]],
                                          sems.at[b4]).wait()

                wait_idx(g, j, b16)
                pltpu.async_copy(p_ref.at[ibs.at[b16]], rows.at[b4],
                                 semg.at[b4])

                @pl.when(j + 6 < NCP)
                def _():
                    fire_idx(g, j + 6, lax.rem(j + 6, 16))

                @pl.when(j >= 3)
                def _():
                    b4p = lax.rem(j + 1, 4)
                    b16p = lax.rem(j + 13, 16)
                    pltpu.make_async_copy(p_ref.at[ibs.at[b16p]],
                                          rows.at[b4p], semg.at[b4p]).wait()
                    pltpu.async_copy(rows.at[b4p], acc.at[ibd.at[b16p]],
                                     sems.at[b4p], add=True)

            for k in (NCP - 3, NCP - 2, NCP - 1):
                pltpu.make_async_copy(p_ref.at[ibs.at[k % 16]],
                                      rows.at[k % 4], semg.at[k % 4]).wait()
                pltpu.async_copy(rows.at[k % 4], acc.at[ibd.at[k % 16]],
                                 sems.at[k % 4], add=True)
            for k in (NCP - 4, NCP - 3, NCP - 2, NCP - 1):
                pltpu.make_async_copy(rows.at[k % 4], acc.at[ibd.at[k % 16]],
                                      sems.at[k % 4]).wait()
            plsc.subcore_barrier()
            pltpu.sync_copy(acc.at[pl.ds(s * RP, RP)],
                            out_ref.at[g, pl.ds(s * RP, RP)])

    return prop


def _tc1(x, w1, deg16):
    def body(x_ref, w_ref, deg_ref, p_ref):
        dinv = lax.rsqrt(deg_ref[0][:, 0:1] + 1.0)
        p_ref[...] = jnp.dot(x_ref[0].astype(jnp.bfloat16),
                             w_ref[0].astype(jnp.bfloat16),
                             preferred_element_type=jnp.float32) * dinv

    return pl.pallas_call(
        body,
        grid=(G,),
        in_specs=[
            pl.BlockSpec((1, N, D), lambda g: (g, 0, 0)),
            pl.BlockSpec((1, D, D), lambda g: (g, 0, 0)),
            pl.BlockSpec((1, N, 16), lambda g: (g, 0, 0)),
        ],
        out_specs=pl.BlockSpec((N, D), lambda g: (g, 0)),
        out_shape=jax.ShapeDtypeStruct((G * N, D), jnp.float32),
    )(x, w1, deg16)


def _tc2(s1, deg16, b1, wml):
    def body(s_ref, deg_ref, b_ref, w_ref, p_ref):
        dinv = lax.rsqrt(deg_ref[0][:, 0:1] + 1.0)
        h = jnp.maximum(s_ref[0] * dinv + b_ref[0], 0.0)
        p_ref[...] = jnp.dot(h.astype(jnp.bfloat16),
                             w_ref[0].astype(jnp.bfloat16),
                             preferred_element_type=jnp.float32) * dinv

    return pl.pallas_call(
        body,
        grid=(G,),
        in_specs=[
            pl.BlockSpec((1, N, D), lambda g: (g, 0, 0)),
            pl.BlockSpec((1, N, 16), lambda g: (g, 0, 0)),
            pl.BlockSpec((1, 1, D), lambda g: (g, 0, 0)),
            pl.BlockSpec((1, D, D), lambda g: (g, 0, 0)),
        ],
        out_specs=pl.BlockSpec((N, D), lambda g: (g, 0)),
        out_shape=jax.ShapeDtypeStruct((G * N, D), jnp.float32),
    )(s1, deg16, b1, wml)


def _tc3(s2, deg16, bml):
    def body(s_ref, deg_ref, b_ref, out_ref):
        dinv = lax.rsqrt(deg_ref[0][:, 0:1] + 1.0)
        t = s_ref[0] * dinv + b_ref[0]
        out_ref[0, 0] = t[:, :OUT]
        out_ref[1, 0] = t[:, OUT:]

    return pl.pallas_call(
        body,
        grid=(G, N // BN),
        in_specs=[
            pl.BlockSpec((1, BN, D), lambda g, i: (g, i, 0)),
            pl.BlockSpec((1, BN, 16), lambda g, i: (g, i, 0)),
            pl.BlockSpec((1, 1, D), lambda g, i: (g, 0, 0)),
        ],
        out_specs=pl.BlockSpec((2, 1, BN, OUT), lambda g, i: (0, g, i, 0)),
        out_shape=jax.ShapeDtypeStruct((2, G, N, OUT), jnp.float32),
    )(s2, deg16, bml)


def kernel(x, edge_index, W1, b1, Wmu, bmu, Wls, bls):
    ei = edge_index.astype(jnp.int32)
    offs = (jnp.arange(G, dtype=jnp.int32) * N)[:, None]
    srco = ei[:, 0, :] + offs
    dstm = ei[:, 1, :].reshape(G, NS, NCH, C)

    zeros16 = jnp.zeros((N, 16), jnp.float32)
    ones16 = jnp.ones((C, 16), jnp.float32)

    hist = _make_hist()
    prop = _make_prop()

    deg16 = hist(dstm, zeros16, ones16)
    p1 = _tc1(x, W1, deg16)
    s1 = prop(p1, srco, ei)
    wml = jnp.concatenate([Wmu, Wls], axis=2)
    bml = jnp.concatenate([bmu, bls], axis=1)[:, None, :]
    p2 = _tc2(s1, deg16, b1[:, None, :], wml)
    s2 = prop(p2, srco, ei)
    return _tc3(s2, deg16, bml)
